# baseline (speedup 1.0000x reference)
import jax
import jax.numpy as jnp
from jax import lax
from jax.experimental import pallas as pl
from jax.experimental.pallas import tpu as pltpu

_NUM_E = 8192
_DIM = 32
_COST = 0.25
_TB = 256
_KB = 512
_CHUNK = 4096


def _vq_body(z_ref, zb_ref, e_ref, qst_ref, loss_ref):
    z_blk = z_ref[...]
    zb_blk = zb_ref[...]
    z_sq = jnp.sum(z_blk ** 2, axis=1, keepdims=True)

    tiles_per_chunk = _CHUNK // _KB

    def chunk_body(c, carry):
        acc_val, acc_idx = carry

        def tile_body(t, tcarry):
            tval, tidx = tcarry
            k0 = c * _CHUNK + t * _KB
            e_tile = e_ref[pl.ds(k0, _KB), :]
            e_sq = jnp.sum(e_tile ** 2, axis=1)
            mm = lax.dot_general(zb_blk, e_tile,
                                 (((1,), (1,)), ((), ())))
            dist = (z_sq + e_sq[None, :]) - 2.0 * mm
            tmin = jnp.min(dist, axis=1, keepdims=True)
            lane = lax.broadcasted_iota(jnp.int32, (_TB, _KB), 1) + k0
            targ = jnp.min(jnp.where(dist == tmin, lane, jnp.int32(2 ** 30)),
                           axis=1, keepdims=True)
            upd = tmin < tval
            return jnp.where(upd, tmin, tval), jnp.where(upd, targ, tidx)

        init = (jnp.full((_TB, 1), jnp.inf, jnp.float32),
                jnp.zeros((_TB, 1), jnp.int32))
        cmin, cidx = lax.fori_loop(0, tiles_per_chunk, tile_body, init)
        upd = cmin < acc_val
        acc_idx = jnp.where(upd, cidx, acc_idx)
        acc_val = jnp.where(upd, cmin, acc_val)
        u = lax.bitcast_convert_type(acc_val, jnp.uint32)
        u = (u + jnp.uint32(0x7FFF) + ((u >> 16) & jnp.uint32(1))) & jnp.uint32(0xFFFF0000)
        acc_val = lax.bitcast_convert_type(u, jnp.float32)
        return acc_val, acc_idx

    init = (jnp.full((_TB, 1), jnp.inf, jnp.float32),
            jnp.zeros((_TB, 1), jnp.int32))
    _, bidx = lax.fori_loop(0, _NUM_E // _CHUNK, chunk_body, init)

    def gather_body(t, acc):
        k0 = t * _KB
        e_tile = e_ref[pl.ds(k0, _KB), :]
        lane = lax.broadcasted_iota(jnp.int32, (_TB, _KB), 1) + k0
        oh = (lane == bidx).astype(jnp.float32)
        return acc + lax.dot_general(oh, e_tile,
                                     (((1,), (0,)), ((), ())))

    q = lax.fori_loop(0, _NUM_E // _KB, gather_body,
                      jnp.zeros((_TB, _DIM), jnp.float32))

    qst_ref[...] = z_blk + (q - z_blk)

    partial = jnp.sum((z_blk - q) ** 2).reshape(1, 1)

    @pl.when(pl.program_id(0) == 0)
    def _():
        loss_ref[...] = jnp.zeros((1, 1), jnp.float32)

    loss_ref[...] += partial


def kernel(z, embedding):
    B, C, H, W = z.shape
    z_flat = jnp.transpose(z, (0, 2, 3, 1)).reshape(-1, C)
    zb_flat = z_flat.astype(jnp.bfloat16).astype(jnp.float32)
    n_tok = z_flat.shape[0]
    grid = n_tok // _TB
    qst_flat, loss_sum = pl.pallas_call(
        _vq_body,
        grid=(grid,),
        in_specs=[pl.BlockSpec((_TB, _DIM), lambda i: (i, 0)),
                  pl.BlockSpec((_TB, _DIM), lambda i: (i, 0)),
                  pl.BlockSpec((_NUM_E, _DIM), lambda i: (0, 0))],
        out_specs=[pl.BlockSpec((_TB, _DIM), lambda i: (i, 0)),
                   pl.BlockSpec((1, 1), lambda i: (0, 0))],
        out_shape=[jax.ShapeDtypeStruct((n_tok, _DIM), jnp.float32),
                   jax.ShapeDtypeStruct((1, 1), jnp.float32)],
    )(z_flat, zb_flat, embedding)
    quantized_st = qst_flat.reshape(B, H, W, C).transpose(0, 3, 1, 2)
    lm = loss_sum[0, 0] / jnp.float32(z.size)
    loss = _COST * lm + lm
    return (quantized_st, loss)
